# baseline (speedup 1.0000x reference)
import functools

import jax
import jax.numpy as jnp
import numpy as np
from jax.experimental import pallas as pl
from jax.experimental.pallas import tpu as pltpu

_N_BINS = 20
_BOUNDS = np.linspace(0.0, 1.0, _N_BINS + 1).astype(np.float32)
_INNER_LOWERS = _BOUNDS[1:_N_BINS]


def _ece_body(x_ref, lab_ref, out_ref, acc_ref, *, n_blocks, n_total):
    i = pl.program_id(0)

    @pl.when(i == 0)
    def _init():
        acc_ref[...] = jnp.zeros_like(acc_ref)

    x = x_ref[...]
    R, C = x.shape
    m = jnp.max(x, axis=1, keepdims=True)
    s = jnp.sum(jnp.exp(x - m), axis=1, keepdims=True)
    conf = 1.0 / s

    lanes = jax.lax.broadcasted_iota(jnp.int32, (R, C), 1)
    pred = jnp.min(jnp.where(x == m, lanes, C), axis=1, keepdims=True)
    lab = lab_ref[0].reshape(R, 1)
    acc = (pred == lab).astype(jnp.float32)

    lowers = (jax.lax.broadcasted_iota(jnp.int32, (1, _N_BINS - 1), 1)
              + 1).astype(jnp.float32) / np.float32(_N_BINS)
    b = jnp.sum((conf > lowers).astype(jnp.int32), axis=1, keepdims=True)
    onehot = (b == jax.lax.broadcasted_iota(jnp.int32, (R, _N_BINS), 1)
              ).astype(jnp.float32)
    cnt = jnp.sum(onehot, axis=0, keepdims=True)
    asum = jnp.sum(onehot * acc, axis=0, keepdims=True)
    csum = jnp.sum(onehot * conf, axis=0, keepdims=True)
    acc_ref[...] += jnp.concatenate([cnt, asum, csum], axis=0)

    @pl.when(i == n_blocks - 1)
    def _fin():
        cntf = acc_ref[0:1, :]
        asumf = acc_ref[1:2, :]
        csumf = acc_ref[2:3, :]
        prop = cntf / np.float32(n_total)
        denom = jnp.maximum(cntf, 1.0)
        contrib = jnp.where(cntf > 0.0,
                            jnp.abs(csumf / denom - asumf / denom) * prop,
                            0.0)
        out_ref[...] = jnp.sum(contrib, axis=1, keepdims=True)


def kernel(logits, labels):
    n, c = logits.shape
    rows = 8000
    n_blocks = n // rows
    labels3 = labels.reshape(n_blocks, 1, rows)

    out = pl.pallas_call(
        functools.partial(_ece_body, n_blocks=n_blocks, n_total=n),
        grid=(n_blocks,),
        in_specs=[
            pl.BlockSpec((rows, c), lambda i: (i, 0)),
            pl.BlockSpec((1, 1, rows), lambda i: (i, 0, 0)),
        ],
        out_specs=pl.BlockSpec((1, 1), lambda i: (0, 0)),
        out_shape=jax.ShapeDtypeStruct((1, 1), jnp.float32),
        scratch_shapes=[pltpu.VMEM((3, _N_BINS), jnp.float32)],
    )(logits, labels3)
    return out.reshape(1)


# speedup vs baseline: 1.1692x; 1.1692x over previous
import functools

import jax
import jax.numpy as jnp
import numpy as np
from jax.experimental import pallas as pl
from jax.experimental.pallas import tpu as pltpu

_N_BINS = 20


def _ece_stage1(x_ref, lab_ref, out_ref, cnt_ref, asum_ref, csum_ref, *,
                nsteps):
    j = pl.program_id(1)

    @pl.when(j == 0)
    def _init():
        cnt_ref[...] = jnp.zeros_like(cnt_ref)
        asum_ref[...] = jnp.zeros_like(asum_ref)
        csum_ref[...] = jnp.zeros_like(csum_ref)

    x = x_ref[...]
    R, C = x.shape
    labc = lab_ref[0].reshape(R, 1)

    m = jnp.max(x, axis=1, keepdims=True)
    s = jnp.sum(jnp.exp(x - m), axis=1, keepdims=True)
    lanes = jax.lax.broadcasted_iota(jnp.int32, (R, C), 1)
    xl = jnp.max(jnp.where(lanes == labc, x, -jnp.inf), axis=1,
                 keepdims=True)
    acc = (xl == m).astype(jnp.float32)

    conf_row = (1.0 / s).T
    acc_row = acc.T

    bounds = (jax.lax.broadcasted_iota(jnp.int32, (_N_BINS, 1), 0)
              ).astype(jnp.float32) / np.float32(_N_BINS)
    g = (conf_row > bounds).astype(jnp.float32)
    gshift = jnp.concatenate(
        [g[1:, :], jnp.zeros((1, R), jnp.float32)], axis=0)
    onehot = g - gshift

    cnt_ref[...] += onehot
    asum_ref[...] += onehot * acc_row
    csum_ref[...] += onehot * conf_row

    @pl.when(j == nsteps - 1)
    def _fin():
        cnt = jnp.sum(cnt_ref[...], axis=1, keepdims=True)
        asum = jnp.sum(asum_ref[...], axis=1, keepdims=True)
        csum = jnp.sum(csum_ref[...], axis=1, keepdims=True)
        out_ref[0] = jnp.concatenate([cnt, asum, csum], axis=1)


def _ece_stage2(p_ref, o_ref, *, n_total):
    tot = p_ref[0] + p_ref[1]
    cnt = tot[:, 0:1]
    asum = tot[:, 1:2]
    csum = tot[:, 2:3]
    prop = cnt / np.float32(n_total)
    denom = jnp.maximum(cnt, 1.0)
    contrib = jnp.where(cnt > 0.0,
                        jnp.abs(csum / denom - asum / denom) * prop,
                        0.0)
    o_ref[...] = jnp.sum(contrib, axis=0, keepdims=True)


def kernel(logits, labels):
    n, c = logits.shape
    rows = 10000
    n_blocks = n // rows
    nsteps = n_blocks // 2
    labels3 = labels.reshape(n_blocks, 1, rows)

    parts = pl.pallas_call(
        functools.partial(_ece_stage1, nsteps=nsteps),
        grid=(2, nsteps),
        in_specs=[
            pl.BlockSpec((rows, c), lambda i, j: (i * nsteps + j, 0)),
            pl.BlockSpec((1, 1, rows), lambda i, j: (i * nsteps + j, 0, 0)),
        ],
        out_specs=pl.BlockSpec((1, _N_BINS, 3), lambda i, j: (i, 0, 0)),
        out_shape=jax.ShapeDtypeStruct((2, _N_BINS, 3), jnp.float32),
        scratch_shapes=[
            pltpu.VMEM((_N_BINS, rows), jnp.float32),
            pltpu.VMEM((_N_BINS, rows), jnp.float32),
            pltpu.VMEM((_N_BINS, rows), jnp.float32),
        ],
        compiler_params=pltpu.CompilerParams(
            dimension_semantics=("parallel", "arbitrary")),
    )(logits, labels3)

    out = pl.pallas_call(
        functools.partial(_ece_stage2, n_total=n),
        out_shape=jax.ShapeDtypeStruct((1, 1), jnp.float32),
    )(parts)
    return out.reshape(1)


# speedup vs baseline: 1.1783x; 1.0078x over previous
import functools

import jax
import jax.numpy as jnp
import numpy as np
from jax.experimental import pallas as pl
from jax.experimental.pallas import tpu as pltpu

_N_BINS = 20


def _half(x, labr):
    H, C = x.shape
    m = jnp.max(x, axis=1, keepdims=True)
    s = jnp.sum(jnp.exp(x - m), axis=1, keepdims=True)
    lanes = jax.lax.broadcasted_iota(jnp.int32, (H, C), 1)
    labc = labr.reshape(H, 1)
    xl = jnp.max(jnp.where(lanes == labc, x, -jnp.inf), axis=1, keepdims=True)
    conf = 1.0 / s
    phi = jnp.where(xl == m, -conf, conf)
    pr = phi.T
    conf_row = jnp.abs(pr)
    acc_row = (pr < 0.0).astype(jnp.float32)
    bounds = (jax.lax.broadcasted_iota(jnp.int32, (_N_BINS, 1), 0)
              ).astype(jnp.float32) / np.float32(_N_BINS)
    g = (conf_row > bounds).astype(jnp.float32)
    gshift = jnp.concatenate(
        [g[1:, :], jnp.zeros((1, H), jnp.float32)], axis=0)
    onehot = g - gshift
    return onehot, acc_row, conf_row


def _ece_stage1(x_ref, lab_ref, out_ref, cnt_ref, asum_ref, csum_ref, *,
                nsteps):
    j = pl.program_id(0)

    @pl.when(j == 0)
    def _init():
        cnt_ref[...] = jnp.zeros_like(cnt_ref)
        asum_ref[...] = jnp.zeros_like(asum_ref)
        csum_ref[...] = jnp.zeros_like(csum_ref)

    x = x_ref[...]
    R, C = x.shape
    H = R // 2
    lab = lab_ref[0]

    oh1, a1, c1 = _half(x[:H, :], lab[:, :H])
    oh2, a2, c2 = _half(x[H:, :], lab[:, H:])

    cnt_ref[:, :H] += oh1
    asum_ref[:, :H] += oh1 * a1
    csum_ref[:, :H] += oh1 * c1
    cnt_ref[:, H:] += oh2
    asum_ref[:, H:] += oh2 * a2
    csum_ref[:, H:] += oh2 * c2

    @pl.when(j == nsteps - 1)
    def _fin():
        cnt = jnp.sum(cnt_ref[...], axis=1, keepdims=True)
        asum = jnp.sum(asum_ref[...], axis=1, keepdims=True)
        csum = jnp.sum(csum_ref[...], axis=1, keepdims=True)
        out_ref[...] = jnp.concatenate([cnt, asum, csum], axis=1)


def _ece_stage2(p_ref, o_ref, *, n_total):
    tot = p_ref[...]
    cnt = tot[:, 0:1]
    asum = tot[:, 1:2]
    csum = tot[:, 2:3]
    prop = cnt / np.float32(n_total)
    denom = jnp.maximum(cnt, 1.0)
    contrib = jnp.where(cnt > 0.0,
                        jnp.abs(csum / denom - asum / denom) * prop,
                        0.0)
    o_ref[...] = jnp.sum(contrib, axis=0, keepdims=True)


def kernel(logits, labels):
    n, c = logits.shape
    rows = 10000
    nsteps = n // rows
    labels3 = labels.reshape(nsteps, 1, rows)

    parts = pl.pallas_call(
        functools.partial(_ece_stage1, nsteps=nsteps),
        grid=(nsteps,),
        in_specs=[
            pl.BlockSpec((rows, c), lambda j: (j, 0)),
            pl.BlockSpec((1, 1, rows), lambda j: (j, 0, 0)),
        ],
        out_specs=pl.BlockSpec((_N_BINS, 3), lambda j: (0, 0)),
        out_shape=jax.ShapeDtypeStruct((_N_BINS, 3), jnp.float32),
        scratch_shapes=[
            pltpu.VMEM((_N_BINS, rows), jnp.float32),
            pltpu.VMEM((_N_BINS, rows), jnp.float32),
            pltpu.VMEM((_N_BINS, rows), jnp.float32),
        ],
    )(logits, labels3)

    out = pl.pallas_call(
        functools.partial(_ece_stage2, n_total=n),
        out_shape=jax.ShapeDtypeStruct((1, 1), jnp.float32),
    )(parts)
    return out.reshape(1)
